# Initial kernel scaffold; baseline (speedup 1.0000x reference)
import jax
import jax.numpy as jnp
from jax.experimental import pallas as pl


def kernel(x, embedding):
    raise NotImplementedError("write your pallas kernel here")


# speedup vs baseline: 1.0939x; 1.0939x over previous
import jax
import jax.numpy as jnp
from jax import lax
from jax.experimental import pallas as pl
from jax.experimental.pallas import tpu as pltpu
from jax.experimental.pallas import tpu_sc as plsc

NUM_ROWS = 16384 * 50
DIM = 32
IDX_MINOR = 128
NC = 2
NS = 16
NW = NC * NS
IR_PER_W = NUM_ROWS // IDX_MINOR // NW
CHUNK_IR = 8
NCHUNK = IR_PER_W // CHUNK_IR
CHUNK_ROWS = CHUNK_IR * IDX_MINOR


def _gather_body(x_hbm, table_hbm, out_hbm, idx_v, rows_v, sem):
    wid = lax.axis_index("s") * NC + lax.axis_index("c")
    ir_base = wid * IR_PER_W

    def body(c, carry):
        ir0 = ir_base + c * CHUNK_IR
        pltpu.sync_copy(x_hbm.at[pl.ds(ir0, CHUNK_IR)], idx_v)
        cps = [
            pltpu.async_copy(
                table_hbm.at[idx_v.at[j]],
                rows_v.at[pl.ds(j * IDX_MINOR, IDX_MINOR)],
                sem,
            )
            for j in range(CHUNK_IR)
        ]
        for cp in cps:
            cp.wait()
        pltpu.sync_copy(rows_v, out_hbm.at[pl.ds(ir0 * IDX_MINOR, CHUNK_ROWS)])
        return carry

    lax.fori_loop(0, NCHUNK, body, None)


def kernel(x, embedding):
    x2 = x.reshape(NUM_ROWS // IDX_MINOR, IDX_MINOR).astype(jnp.int32)
    mesh = plsc.VectorSubcoreMesh(core_axis_name="c", subcore_axis_name="s")
    out = pl.kernel(
        _gather_body,
        mesh=mesh,
        compiler_params=pltpu.CompilerParams(use_tc_tiling_on_sc=False),
        out_type=jax.ShapeDtypeStruct((NUM_ROWS, DIM), jnp.float32),
        scratch_types=[
            pltpu.VMEM((CHUNK_IR, IDX_MINOR), jnp.int32),
            pltpu.VMEM((CHUNK_ROWS, DIM), jnp.float32),
            pltpu.SemaphoreType.DMA,
        ],
    )(x2, embedding)
    return out.reshape(16384, 50, DIM)


# speedup vs baseline: 1.1127x; 1.0171x over previous
import jax
import jax.numpy as jnp
from jax import lax
from jax.experimental import pallas as pl
from jax.experimental.pallas import tpu as pltpu
from jax.experimental.pallas import tpu_sc as plsc

NUM_ROWS = 16384 * 50
DIM = 32
IDX_MINOR = 128
NC = 2
NS = 16
NW = NC * NS
IR_PER_W = NUM_ROWS // IDX_MINOR // NW
CHUNK_IR = 4
NCHUNK = IR_PER_W // CHUNK_IR
CHUNK_ROWS = CHUNK_IR * IDX_MINOR
NBUF = 3


def _gather_body(x_hbm, table_hbm, out_hbm,
                 idx_v, rows0, rows1, rows2,
                 gsem0, gsem1, gsem2, ssem0, ssem1, ssem2):
    rows = (rows0, rows1, rows2)
    gsem = (gsem0, gsem1, gsem2)
    ssem = (ssem0, ssem1, ssem2)

    wid = lax.axis_index("s") * NC + lax.axis_index("c")
    ir_base = wid * IR_PER_W
    row_base = ir_base * IDX_MINOR

    pltpu.sync_copy(x_hbm.at[pl.ds(ir_base, IR_PER_W)], idx_v)

    def fire_gathers(c, b):
        for j in range(CHUNK_IR):
            pltpu.async_copy(
                table_hbm.at[idx_v.at[c * CHUNK_IR + j]],
                rows[b].at[pl.ds(j * IDX_MINOR, IDX_MINOR)],
                gsem[b],
            )

    def wait_gathers(b):
        pltpu.make_async_copy(
            table_hbm.at[pl.ds(0, CHUNK_ROWS)], rows[b], gsem[b]
        ).wait()

    def fire_store(c, b):
        pltpu.async_copy(
            rows[b],
            out_hbm.at[pl.ds(row_base + c * CHUNK_ROWS, CHUNK_ROWS)],
            ssem[b],
        )

    def wait_store(b):
        pltpu.make_async_copy(
            rows[b], out_hbm.at[pl.ds(0, CHUNK_ROWS)], ssem[b]
        ).wait()

    fire_gathers(0, 0)
    fire_gathers(1, 1)
    wait_gathers(0)
    fire_store(0, 0)
    fire_gathers(2, 2)
    wait_gathers(1)
    fire_store(1, 1)

    def group(g, carry):
        for k in range(NBUF):
            c = NBUF + g * NBUF + k
            prev = (k + NBUF - 1) % NBUF
            wait_store(k)
            fire_gathers(c, k)
            wait_gathers(prev)
            fire_store(c - 1, prev)
        return carry

    ngroups = (NCHUNK - NBUF) // NBUF
    lax.fori_loop(0, ngroups, group, None)

    for c in range(NBUF + ngroups * NBUF, NCHUNK):
        b = c % NBUF
        prev = (b + NBUF - 1) % NBUF
        wait_store(b)
        fire_gathers(c, b)
        wait_gathers(prev)
        fire_store(c - 1, prev)

    last = (NCHUNK - 1) % NBUF
    wait_gathers(last)
    fire_store(NCHUNK - 1, last)
    for b in range(NBUF):
        wait_store(b)


def kernel(x, embedding):
    x2 = x.reshape(NUM_ROWS // IDX_MINOR, IDX_MINOR).astype(jnp.int32)
    mesh = plsc.VectorSubcoreMesh(core_axis_name="c", subcore_axis_name="s")
    out = pl.kernel(
        _gather_body,
        mesh=mesh,
        compiler_params=pltpu.CompilerParams(use_tc_tiling_on_sc=False),
        out_type=jax.ShapeDtypeStruct((NUM_ROWS, DIM), jnp.float32),
        scratch_types=[
            pltpu.VMEM((IR_PER_W, IDX_MINOR), jnp.int32),
            pltpu.VMEM((CHUNK_ROWS, DIM), jnp.float32),
            pltpu.VMEM((CHUNK_ROWS, DIM), jnp.float32),
            pltpu.VMEM((CHUNK_ROWS, DIM), jnp.float32),
            pltpu.SemaphoreType.DMA,
            pltpu.SemaphoreType.DMA,
            pltpu.SemaphoreType.DMA,
            pltpu.SemaphoreType.DMA,
            pltpu.SemaphoreType.DMA,
            pltpu.SemaphoreType.DMA,
        ],
    )(x2, embedding)
    return out.reshape(16384, 50, DIM)


# speedup vs baseline: 1.8045x; 1.6218x over previous
import jax
import jax.numpy as jnp
from jax import lax
from jax.experimental import pallas as pl
from jax.experimental.pallas import tpu as pltpu
from jax.experimental.pallas import tpu_sc as plsc

B = 16384
S = 50
DIM = 32
NC = 2
NS = 16
NW = NC * NS
BR_PER_W = B // NW
CHUNK_BR = 8
NCHUNK = BR_PER_W // CHUNK_BR
NBUF = 3


def _gather_body(x_hbm, table_hbm, out_hbm,
                 idx_v, rows0, rows1, rows2,
                 gsem0, gsem1, gsem2, ssem0, ssem1, ssem2):
    rows = (rows0, rows1, rows2)
    gsem = (gsem0, gsem1, gsem2)
    ssem = (ssem0, ssem1, ssem2)

    wid = lax.axis_index("s") * NC + lax.axis_index("c")
    br_base = wid * BR_PER_W

    pltpu.sync_copy(x_hbm.at[pl.ds(br_base, BR_PER_W)], idx_v)

    def fire_gathers(c, b):
        for j in range(CHUNK_BR):
            pltpu.async_copy(
                table_hbm.at[idx_v.at[c * CHUNK_BR + j]],
                rows[b].at[j],
                gsem[b],
            )

    def wait_gathers(b):
        pltpu.make_async_copy(
            out_hbm.at[pl.ds(0, CHUNK_BR)], rows[b], gsem[b]
        ).wait()

    def fire_store(c, b):
        pltpu.async_copy(
            rows[b],
            out_hbm.at[pl.ds(br_base + c * CHUNK_BR, CHUNK_BR)],
            ssem[b],
        )

    def wait_store(b):
        pltpu.make_async_copy(
            rows[b], out_hbm.at[pl.ds(0, CHUNK_BR)], ssem[b]
        ).wait()

    fire_gathers(0, 0)
    fire_gathers(1, 1)
    wait_gathers(0)
    fire_store(0, 0)
    fire_gathers(2, 2)
    wait_gathers(1)
    fire_store(1, 1)

    def group(g, carry):
        for k in range(NBUF):
            c = NBUF + g * NBUF + k
            prev = (k + NBUF - 1) % NBUF
            wait_store(k)
            fire_gathers(c, k)
            wait_gathers(prev)
            fire_store(c - 1, prev)
        return carry

    ngroups = (NCHUNK - NBUF) // NBUF
    lax.fori_loop(0, ngroups, group, None)

    for c in range(NBUF + ngroups * NBUF, NCHUNK):
        b = c % NBUF
        prev = (b + NBUF - 1) % NBUF
        wait_store(b)
        fire_gathers(c, b)
        wait_gathers(prev)
        fire_store(c - 1, prev)

    last = (NCHUNK - 1) % NBUF
    wait_gathers(last)
    fire_store(NCHUNK - 1, last)
    for b in range(NBUF):
        wait_store(b)


def kernel(x, embedding):
    xi = x.astype(jnp.int32)
    mesh = plsc.VectorSubcoreMesh(core_axis_name="c", subcore_axis_name="s")
    out = pl.kernel(
        _gather_body,
        mesh=mesh,
        compiler_params=pltpu.CompilerParams(use_tc_tiling_on_sc=False),
        out_type=jax.ShapeDtypeStruct((B, S, DIM), jnp.float32),
        scratch_types=[
            pltpu.VMEM((BR_PER_W, S), jnp.int32),
            pltpu.VMEM((CHUNK_BR, S, DIM), jnp.float32),
            pltpu.VMEM((CHUNK_BR, S, DIM), jnp.float32),
            pltpu.VMEM((CHUNK_BR, S, DIM), jnp.float32),
            pltpu.SemaphoreType.DMA,
            pltpu.SemaphoreType.DMA,
            pltpu.SemaphoreType.DMA,
            pltpu.SemaphoreType.DMA,
            pltpu.SemaphoreType.DMA,
            pltpu.SemaphoreType.DMA,
        ],
    )(xi, embedding)
    return out
